# baseline (speedup 1.0000x reference)
import functools

import jax
import jax.numpy as jnp
from jax import lax
from jax.experimental import pallas as pl
from jax.experimental.pallas import tpu as pltpu
from jax.experimental.pallas import tpu_sc as plsc

N = 10000
NP = 10112
E = 320000
CH = 128
NC = 2
NS = 16
NW = NC * NS
RPT = NP // NS
GRP = 8

MCH = 112
MNG = 12
MRPW = MNG * GRP
MEP = NW * MRPW * MCH

DCH = 128
DROWS = 2560
DRPW = DROWS // NW
DNG = DRPW // GRP
DEP = DROWS * DCH

_mesh = plsc.VectorSubcoreMesh(core_axis_name="c", subcore_axis_name="s")

M_PIECES = ((0, 112), (112, 112), (224, 112), (336, 112), (448, 112), (560, 72))
D_PIECES = ((0, 128), (128, 128), (256, 128), (384, 128), (512, 120))


def _zero_fill(buf, rows):
    def zrow(i, _):
        for j in range(CH // 16):
            buf[i, pl.ds(j * 16, 16)] = jnp.zeros((16,), jnp.float32)
        return 0

    lax.fori_loop(0, rows, zrow, 0)


def _acc_init(buf, acc_sh, s, pieces):
    for off, h in pieces:
        pltpu.sync_copy(buf.at[pl.ds(0, h)], acc_sh.at[pl.ds(s * RPT + off, h)])


def _acc_readback(buf, acc_sh, s, c, p0_hbm, p1_hbm, pieces):
    for off, h in pieces:
        pltpu.sync_copy(acc_sh.at[pl.ds(s * RPT + off, h)], buf.at[pl.ds(0, h)])

        @pl.when(c == 0)
        def _():
            pltpu.sync_copy(buf.at[pl.ds(0, h)], p0_hbm.at[s, pl.ds(off, h)])

        @pl.when(c == 1)
        def _():
            pltpu.sync_copy(buf.at[pl.ds(0, h)], p1_hbm.at[s, pl.ds(off, h)])


_OUT2 = (
    jax.ShapeDtypeStruct((NS, RPT, CH), jnp.float32),
    jax.ShapeDtypeStruct((NS, RPT, CH), jnp.float32),
)


@functools.partial(
    pl.kernel,
    out_type=_OUT2,
    mesh=_mesh,
    scratch_types=(
        [pltpu.VMEM((GRP, MCH), jnp.int32)] * 6
        + [pltpu.VMEM((MCH, CH), jnp.float32)] * 3
        + [pltpu.VMEM_SHARED((NP, CH), jnp.float32)]
        + [pltpu.SemaphoreType.DMA] * 9
    ),
)
def _sc_scatter(g_hbm, src_hbm, dst_hbm, p0_hbm, p1_hbm,
                is0, is1, is2, id0, id1, id2, r0, r1, r2, acc_sh,
                gs0, gs1, gs2, ss0, ss1, ss2, ix0, ix1, ix2):
    c = lax.axis_index("c")
    s = lax.axis_index("s")
    wid = s * NC + c
    ebase = wid * (MRPW * MCH)
    ISL, DSL = (is0, is1, is2), (id0, id1, id2)
    RB = (r0, r1, r2)
    GSEM, SSEM, ISEM = (gs0, gs1, gs2), (ss0, ss1, ss2), (ix0, ix1, ix2)

    def idx_start(slot, g_idx):
        base = ebase + g_idx * (GRP * MCH)
        for j in range(GRP):
            off = base + j * MCH
            pltpu.make_async_copy(
                src_hbm.at[pl.ds(off, MCH)], ISL[slot].at[j], ISEM[slot]).start()
            pltpu.make_async_copy(
                dst_hbm.at[pl.ds(off, MCH)], DSL[slot].at[j], ISEM[slot]).start()

    def idx_wait(slot):
        for j in range(GRP):
            pltpu.make_async_copy(
                src_hbm.at[pl.ds(0, MCH)], ISL[slot].at[j], ISEM[slot]).wait()
            pltpu.make_async_copy(
                dst_hbm.at[pl.ds(0, MCH)], DSL[slot].at[j], ISEM[slot]).wait()

    def g_start(slot, j, b):
        pltpu.make_async_copy(g_hbm.at[ISL[slot].at[j]], RB[b], GSEM[b]).start()

    def g_wait(slot, j, b):
        pltpu.make_async_copy(g_hbm.at[ISL[slot].at[j]], RB[b], GSEM[b]).wait()

    def sc_start(slot, j, b):
        pltpu.make_async_copy(
            RB[b], acc_sh.at[DSL[slot].at[j]], SSEM[b]).start(add=True)

    def sc_wait(b):
        pltpu.make_async_copy(RB[b], acc_sh.at[DSL[0].at[0]], SSEM[b]).wait()

    idx_start(0, 0)
    _zero_fill(r0, MCH)
    _acc_init(r0, acc_sh, s, M_PIECES)
    plsc.subcore_barrier()

    def triple(t, _):
        for q, (slot, p) in enumerate(((0, 0), (1, 2), (2, 1))):
            g_idx = t * 3 + q
            idx_wait(slot)
            for j in range(GRP):
                b = (p + j) % 3
                if q == 0 and j < 3:
                    @pl.when(t > 0)
                    def _():
                        sc_wait(b)
                else:
                    sc_wait(b)
                g_start(slot, j, b)
                pb = (p + j - 1) % 3
                pslot, pj = (slot, j - 1) if j > 0 else ((slot + 2) % 3, GRP - 1)
                if q == 0 and j == 0:
                    @pl.when(t > 0)
                    def _():
                        g_wait(pslot, pj, pb)
                        sc_start(pslot, pj, pb)
                else:
                    g_wait(pslot, pj, pb)
                    sc_start(pslot, pj, pb)
                if j == 3:
                    @pl.when(g_idx + 1 < MNG)
                    def _():
                        idx_start((slot + 1) % 3, g_idx + 1)
        return 0

    lax.fori_loop(0, MNG // 3, triple, 0)
    g_wait(2, GRP - 1, 2)
    sc_start(2, GRP - 1, 2)
    sc_wait(0)
    sc_wait(1)
    sc_wait(2)
    plsc.subcore_barrier()
    _acc_readback(r0, acc_sh, s, c, p0_hbm, p1_hbm, M_PIECES)


@functools.partial(
    pl.kernel,
    out_type=_OUT2,
    mesh=_mesh,
    scratch_types=(
        [pltpu.VMEM((GRP, DCH), jnp.int32)] * 2
        + [pltpu.VMEM((DCH, CH), jnp.float32)] * 2
        + [pltpu.VMEM_SHARED((NP, CH), jnp.float32)]
        + [pltpu.SemaphoreType.DMA] * 4
    ),
)
def _sc_degree(dst_hbm, p0_hbm, p1_hbm,
               id0, id1, ones_v, buf_v, acc_sh, ss0, ss1, ix0, ix1):
    c = lax.axis_index("c")
    s = lax.axis_index("s")
    wid = s * NC + c
    row0 = wid * DRPW
    DSL, SSEM, ISEM = (id0, id1), (ss0, ss1), (ix0, ix1)

    def idx_start(slot, g_idx):
        pltpu.make_async_copy(
            dst_hbm.at[pl.ds(row0 + g_idx * GRP, GRP)], DSL[slot],
            ISEM[slot]).start()

    def idx_wait(slot):
        pltpu.make_async_copy(
            dst_hbm.at[pl.ds(0, GRP)], DSL[slot], ISEM[slot]).wait()

    def sc_start(slot, j, b):
        pltpu.make_async_copy(
            ones_v, acc_sh.at[DSL[slot].at[j]], SSEM[b]).start(add=True)

    def sc_wait(b):
        pltpu.make_async_copy(
            ones_v, acc_sh.at[DSL[0].at[0]], SSEM[b]).wait()

    idx_start(0, 0)
    _zero_fill(buf_v, DCH)
    _acc_init(buf_v, acc_sh, s, D_PIECES)

    def onerow(i, _):
        for j in range(CH // 16):
            ones_v[i, pl.ds(j * 16, 16)] = jnp.ones((16,), jnp.float32)
        return 0

    lax.fori_loop(0, DCH, onerow, 0)
    plsc.subcore_barrier()

    def group_pair(gp, _):
        for slot in (0, 1):
            g_idx = gp * 2 + slot
            idx_wait(slot)
            for j in range(GRP):
                b = j % 2
                if j >= 2:
                    sc_wait(b)
                else:
                    @pl.when(g_idx > 0)
                    def _():
                        sc_wait(b)
                sc_start(slot, j, b)
                if j == 2:
                    @pl.when(g_idx + 1 < DNG)
                    def _():
                        idx_start(1 - slot, g_idx + 1)
        return 0

    lax.fori_loop(0, DNG // 2, group_pair, 0)
    sc_wait(0)
    sc_wait(1)
    plsc.subcore_barrier()

    _acc_readback(buf_v, acc_sh, s, c, p0_hbm, p1_hbm, D_PIECES)


BLK = 2000
GRID = N // BLK


def _row_spec(w):
    return pl.BlockSpec((BLK, w), lambda i: (i, 0))


def _full_spec(h, w):
    return pl.BlockSpec((h, w), lambda i: (0, 0))


def _prep_body(x_ref, w1_ref, d0_ref, d1_ref, g1_ref, dinv_ref):
    deg = d0_ref[:, 0:1] + d1_ref[:, 0:1] + 1.0
    dinv = lax.rsqrt(deg)
    h = jnp.dot(x_ref[...], w1_ref[...], preferred_element_type=jnp.float32)
    g1_ref[...] = h * dinv
    dinv_ref[...] = dinv


_prep = pl.pallas_call(
    _prep_body,
    grid=(GRID,),
    in_specs=[_row_spec(CH), _full_spec(CH, CH), _row_spec(CH), _row_spec(CH)],
    out_specs=(_row_spec(CH), _row_spec(1)),
    out_shape=(
        jax.ShapeDtypeStruct((N, CH), jnp.float32),
        jax.ShapeDtypeStruct((N, 1), jnp.float32),
    ),
)


def _mid_body(p0_ref, p1_ref, g1_ref, dinv_ref, b1_ref, w2_ref, g2_ref):
    dinv = dinv_ref[...]
    h1 = dinv * (p0_ref[...] + p1_ref[...] + g1_ref[...]) + b1_ref[...]
    h1 = jnp.maximum(h1, 0.0)
    g2_ref[...] = dinv * jnp.dot(h1, w2_ref[...], preferred_element_type=jnp.float32)


_mid = pl.pallas_call(
    _mid_body,
    grid=(GRID,),
    in_specs=[_row_spec(CH), _row_spec(CH), _row_spec(CH), _row_spec(1),
              pl.BlockSpec((CH,), lambda i: (0,)), _full_spec(CH, CH)],
    out_specs=_row_spec(CH),
    out_shape=jax.ShapeDtypeStruct((N, CH), jnp.float32),
)


def _head_body(p0_ref, p1_ref, g2_ref, dinv_ref, b2_ref, wl_ref, bl_ref, o_ref):
    h2 = dinv_ref[...] * (p0_ref[...] + p1_ref[...] + g2_ref[...]) + b2_ref[...]
    z = jnp.dot(h2, wl_ref[...], preferred_element_type=jnp.float32) + bl_ref[...]
    o_ref[...] = 1.0 / (1.0 + jnp.exp(-z))


_head = pl.pallas_call(
    _head_body,
    grid=(GRID,),
    in_specs=[_row_spec(CH), _row_spec(CH), _row_spec(CH), _row_spec(1),
              pl.BlockSpec((CH,), lambda i: (0,)), _full_spec(CH, 1),
              pl.BlockSpec((1,), lambda i: (0,))],
    out_specs=_row_spec(1),
    out_shape=jax.ShapeDtypeStruct((N, 1), jnp.float32),
)


def _unpad(p):
    return p.reshape(NP, CH)[:N]


def kernel(x, edge_index, W1, b1, W2, b2, Wl, bl):
    src = edge_index[0].astype(jnp.int32)
    dst = edge_index[1].astype(jnp.int32)

    mpad = jnp.arange(MEP - E, dtype=jnp.int32)
    srcp = jnp.concatenate([src, mpad % N])
    dstp = jnp.concatenate([dst, N + mpad % (NP - N)])

    dpad = jnp.arange(DEP - E, dtype=jnp.int32)
    dstp2 = jnp.concatenate([dst, N + dpad % (NP - N)]).reshape(DROWS, DCH)

    d0, d1 = _sc_degree(dstp2)
    g1, dinv128 = _prep(x, W1, _unpad(d0), _unpad(d1))
    a0, a1 = _sc_scatter(g1, srcp, dstp)
    g2 = _mid(_unpad(a0), _unpad(a1), g1, dinv128, b1, W2)
    c0, c1 = _sc_scatter(g2, srcp, dstp)
    return _head(_unpad(c0), _unpad(c1), g2, dinv128, b2, Wl, bl)


# speedup vs baseline: 1.0017x; 1.0017x over previous
import functools

import jax
import jax.numpy as jnp
from jax import lax
from jax.experimental import pallas as pl
from jax.experimental.pallas import tpu as pltpu
from jax.experimental.pallas import tpu_sc as plsc

N = 10000
NP = 10112
E = 320000
CH = 128
NC = 2
NS = 16
NW = NC * NS
RPT = NP // NS
GRP = 8

MCH = 112
MNG = 12
MRPW = MNG * GRP
MEP = NW * MRPW * MCH

DCH = 128
DROWS = 2560
DRPW = DROWS // NW
DNG = DRPW // GRP
DEP = DROWS * DCH

_mesh = plsc.VectorSubcoreMesh(core_axis_name="c", subcore_axis_name="s")

M_PIECES = ((0, 112), (112, 112), (224, 112), (336, 112), (448, 112), (560, 72))
D_PIECES = ((0, 128), (128, 128), (256, 128), (384, 128), (512, 120))


def _zero_fill(buf, rows):
    def zrow(i, _):
        for j in range(CH // 16):
            buf[i, pl.ds(j * 16, 16)] = jnp.zeros((16,), jnp.float32)
        return 0

    lax.fori_loop(0, rows, zrow, 0)


def _acc_init(buf, acc_sh, s, pieces):
    for off, h in pieces:
        pltpu.sync_copy(buf.at[pl.ds(0, h)], acc_sh.at[pl.ds(s * RPT + off, h)])


def _acc_readback(buf, acc_sh, s, c, p0_hbm, p1_hbm, pieces):
    for off, h in pieces:
        pltpu.sync_copy(acc_sh.at[pl.ds(s * RPT + off, h)], buf.at[pl.ds(0, h)])

        @pl.when(c == 0)
        def _():
            pltpu.sync_copy(buf.at[pl.ds(0, h)], p0_hbm.at[s, pl.ds(off, h)])

        @pl.when(c == 1)
        def _():
            pltpu.sync_copy(buf.at[pl.ds(0, h)], p1_hbm.at[s, pl.ds(off, h)])


_OUT2 = (
    jax.ShapeDtypeStruct((NS, RPT, CH), jnp.float32),
    jax.ShapeDtypeStruct((NS, RPT, CH), jnp.float32),
)


@functools.partial(
    pl.kernel,
    out_type=_OUT2,
    mesh=_mesh,
    scratch_types=(
        [pltpu.VMEM((GRP, MCH), jnp.int32)] * 6
        + [pltpu.VMEM((MCH, CH), jnp.float32)] * 3
        + [pltpu.VMEM_SHARED((NP, CH), jnp.float32)]
        + [pltpu.SemaphoreType.DMA] * 9
    ),
)
def _sc_scatter(g_hbm, src_hbm, dst_hbm, p0_hbm, p1_hbm,
                is0, is1, is2, id0, id1, id2, r0, r1, r2, acc_sh,
                gs0, gs1, gs2, ss0, ss1, ss2, ix0, ix1, ix2):
    c = lax.axis_index("c")
    s = lax.axis_index("s")
    wid = s * NC + c
    ebase = wid * (MRPW * MCH)
    ISL, DSL = (is0, is1, is2), (id0, id1, id2)
    RB = (r0, r1, r2)
    GSEM, SSEM, ISEM = (gs0, gs1, gs2), (ss0, ss1, ss2), (ix0, ix1, ix2)

    def idx_start(slot, g_idx):
        base = ebase + g_idx * (GRP * MCH)
        for j in range(GRP):
            off = base + j * MCH
            pltpu.make_async_copy(
                src_hbm.at[pl.ds(off, MCH)], ISL[slot].at[j], ISEM[slot]).start()
            pltpu.make_async_copy(
                dst_hbm.at[pl.ds(off, MCH)], DSL[slot].at[j], ISEM[slot]).start()

    def idx_wait(slot):
        for j in range(GRP):
            pltpu.make_async_copy(
                src_hbm.at[pl.ds(0, MCH)], ISL[slot].at[j], ISEM[slot]).wait()
            pltpu.make_async_copy(
                dst_hbm.at[pl.ds(0, MCH)], DSL[slot].at[j], ISEM[slot]).wait()

    def g_start(slot, j, b):
        pltpu.make_async_copy(g_hbm.at[ISL[slot].at[j]], RB[b], GSEM[b]).start()

    def g_wait(slot, j, b):
        pltpu.make_async_copy(g_hbm.at[ISL[slot].at[j]], RB[b], GSEM[b]).wait()

    def sc_start(slot, j, b):
        pltpu.make_async_copy(
            RB[b], acc_sh.at[DSL[slot].at[j]], SSEM[b]).start(add=True)

    def sc_wait(b):
        pltpu.make_async_copy(RB[b], acc_sh.at[DSL[0].at[0]], SSEM[b]).wait()

    idx_start(0, 0)
    _zero_fill(r0, MCH)
    _acc_init(r0, acc_sh, s, M_PIECES)
    plsc.subcore_barrier()

    def triple(t, _):
        for q, (slot, p) in enumerate(((0, 0), (1, 2), (2, 1))):
            g_idx = t * 3 + q
            idx_wait(slot)
            for j in range(GRP):
                b = (p + j) % 3
                if q == 0 and j < 3:
                    @pl.when(t > 0)
                    def _():
                        sc_wait(b)
                else:
                    sc_wait(b)
                g_start(slot, j, b)
                pb = (p + j - 1) % 3
                pslot, pj = (slot, j - 1) if j > 0 else ((slot + 2) % 3, GRP - 1)
                if q == 0 and j == 0:
                    @pl.when(t > 0)
                    def _():
                        g_wait(pslot, pj, pb)
                        sc_start(pslot, pj, pb)
                else:
                    g_wait(pslot, pj, pb)
                    sc_start(pslot, pj, pb)
                if j == 3:
                    @pl.when(g_idx + 1 < MNG)
                    def _():
                        idx_start((slot + 1) % 3, g_idx + 1)
        return 0

    lax.fori_loop(0, MNG // 3, triple, 0)
    g_wait(2, GRP - 1, 2)
    sc_start(2, GRP - 1, 2)
    sc_wait(0)
    sc_wait(1)
    sc_wait(2)
    plsc.subcore_barrier()
    _acc_readback(r0, acc_sh, s, c, p0_hbm, p1_hbm, M_PIECES)


@functools.partial(
    pl.kernel,
    out_type=_OUT2,
    mesh=_mesh,
    scratch_types=(
        [pltpu.VMEM((GRP, DCH), jnp.int32)] * 2
        + [pltpu.VMEM((DCH, CH), jnp.float32)] * 2
        + [pltpu.VMEM_SHARED((NP, CH), jnp.float32)]
        + [pltpu.SemaphoreType.DMA] * 4
    ),
)
def _sc_degree(dst_hbm, p0_hbm, p1_hbm,
               id0, id1, ones_v, buf_v, acc_sh, ss0, ss1, ix0, ix1):
    c = lax.axis_index("c")
    s = lax.axis_index("s")
    wid = s * NC + c
    row0 = wid * DRPW
    DSL, SSEM, ISEM = (id0, id1), (ss0, ss1), (ix0, ix1)

    def idx_start(slot, g_idx):
        pltpu.make_async_copy(
            dst_hbm.at[pl.ds(row0 + g_idx * GRP, GRP)], DSL[slot],
            ISEM[slot]).start()

    def idx_wait(slot):
        pltpu.make_async_copy(
            dst_hbm.at[pl.ds(0, GRP)], DSL[slot], ISEM[slot]).wait()

    def sc_start(slot, j, b):
        pltpu.make_async_copy(
            ones_v, acc_sh.at[DSL[slot].at[j]], SSEM[b]).start(add=True)

    def sc_wait(b):
        pltpu.make_async_copy(
            ones_v, acc_sh.at[DSL[0].at[0]], SSEM[b]).wait()

    idx_start(0, 0)
    _zero_fill(buf_v, DCH)
    _acc_init(buf_v, acc_sh, s, D_PIECES)

    def onerow(i, _):
        for j in range(CH // 16):
            ones_v[i, pl.ds(j * 16, 16)] = jnp.ones((16,), jnp.float32)
        return 0

    lax.fori_loop(0, DCH, onerow, 0)
    plsc.subcore_barrier()

    def group_pair(gp, _):
        for slot in (0, 1):
            g_idx = gp * 2 + slot
            idx_wait(slot)
            for j in range(GRP):
                b = j % 2
                if j >= 2:
                    sc_wait(b)
                else:
                    @pl.when(g_idx > 0)
                    def _():
                        sc_wait(b)
                sc_start(slot, j, b)
                if j == 2:
                    @pl.when(g_idx + 1 < DNG)
                    def _():
                        idx_start(1 - slot, g_idx + 1)
        return 0

    lax.fori_loop(0, DNG // 2, group_pair, 0)
    sc_wait(0)
    sc_wait(1)
    plsc.subcore_barrier()

    _acc_readback(buf_v, acc_sh, s, c, p0_hbm, p1_hbm, D_PIECES)


BLK = 2000
GRID = N // BLK


def _row_spec(w):
    return pl.BlockSpec((BLK, w), lambda i: (i, 0))


def _full_spec(h, w):
    return pl.BlockSpec((h, w), lambda i: (0, 0))


def _mm1_body(x_ref, w1_ref, h1_ref):
    h1_ref[...] = jnp.dot(x_ref[...], w1_ref[...],
                          preferred_element_type=jnp.float32)


_mm1 = pl.pallas_call(
    _mm1_body,
    grid=(GRID,),
    in_specs=[_row_spec(CH), _full_spec(CH, CH)],
    out_specs=_row_spec(CH),
    out_shape=jax.ShapeDtypeStruct((N, CH), jnp.float32),
)


def _prep_body(h1_ref, d0_ref, d1_ref, g1_ref, dinv_ref):
    deg = d0_ref[:, 0:1] + d1_ref[:, 0:1] + 1.0
    dinv = lax.rsqrt(deg)
    g1_ref[...] = h1_ref[...] * dinv
    dinv_ref[...] = dinv


_prep = pl.pallas_call(
    _prep_body,
    grid=(GRID,),
    in_specs=[_row_spec(CH), _row_spec(CH), _row_spec(CH)],
    out_specs=(_row_spec(CH), _row_spec(1)),
    out_shape=(
        jax.ShapeDtypeStruct((N, CH), jnp.float32),
        jax.ShapeDtypeStruct((N, 1), jnp.float32),
    ),
)


def _mid_body(p0_ref, p1_ref, g1_ref, dinv_ref, b1_ref, w2_ref, g2_ref):
    dinv = dinv_ref[...]
    h1 = dinv * (p0_ref[...] + p1_ref[...] + g1_ref[...]) + b1_ref[...]
    h1 = jnp.maximum(h1, 0.0)
    g2_ref[...] = dinv * jnp.dot(h1, w2_ref[...], preferred_element_type=jnp.float32)


_mid = pl.pallas_call(
    _mid_body,
    grid=(GRID,),
    in_specs=[_row_spec(CH), _row_spec(CH), _row_spec(CH), _row_spec(1),
              pl.BlockSpec((CH,), lambda i: (0,)), _full_spec(CH, CH)],
    out_specs=_row_spec(CH),
    out_shape=jax.ShapeDtypeStruct((N, CH), jnp.float32),
)


def _head_body(p0_ref, p1_ref, g2_ref, dinv_ref, b2_ref, wl_ref, bl_ref, o_ref):
    h2 = dinv_ref[...] * (p0_ref[...] + p1_ref[...] + g2_ref[...]) + b2_ref[...]
    z = jnp.dot(h2, wl_ref[...], preferred_element_type=jnp.float32) + bl_ref[...]
    o_ref[...] = 1.0 / (1.0 + jnp.exp(-z))


_head = pl.pallas_call(
    _head_body,
    grid=(GRID,),
    in_specs=[_row_spec(CH), _row_spec(CH), _row_spec(CH), _row_spec(1),
              pl.BlockSpec((CH,), lambda i: (0,)), _full_spec(CH, 1),
              pl.BlockSpec((1,), lambda i: (0,))],
    out_specs=_row_spec(1),
    out_shape=jax.ShapeDtypeStruct((N, 1), jnp.float32),
)


def _unpad(p):
    return p.reshape(NP, CH)[:N]


def kernel(x, edge_index, W1, b1, W2, b2, Wl, bl):
    src = edge_index[0].astype(jnp.int32)
    dst = edge_index[1].astype(jnp.int32)

    mpad = jnp.arange(MEP - E, dtype=jnp.int32)
    srcp = jnp.concatenate([src, mpad % N])
    dstp = jnp.concatenate([dst, N + mpad % (NP - N)])

    dpad = jnp.arange(DEP - E, dtype=jnp.int32)
    dstp2 = jnp.concatenate([dst, N + dpad % (NP - N)]).reshape(DROWS, DCH)

    h1 = _mm1(x, W1)
    d0, d1 = _sc_degree(dstp2)
    g1, dinv128 = _prep(h1, _unpad(d0), _unpad(d1))
    a0, a1 = _sc_scatter(g1, srcp, dstp)
    g2 = _mid(_unpad(a0), _unpad(a1), g1, dinv128, b1, W2)
    c0, c1 = _sc_scatter(g2, srcp, dstp)
    return _head(_unpad(c0), _unpad(c1), g2, dinv128, b2, Wl, bl)
